# baseline (speedup 1.0000x reference)
import functools

import jax
import jax.numpy as jnp
from jax import lax
from jax.experimental import pallas as pl
from jax.experimental.pallas import tpu as pltpu
from jax.experimental.pallas import tpu_sc as plsc

_THRESH = 0.35667494393873245
_N_MIN = 16 * 512 * 512 // 16
_NT = 16
_CHUNK = 32768


def _ce_loss(logits_ref, labels_ref):
    x = logits_ref[0]
    lab = labels_ref[0]
    m = jnp.max(x, axis=0)
    s = jnp.sum(jnp.exp(x - m[None, :, :]), axis=0)
    lse = m + jnp.log(s)
    c = jax.lax.broadcasted_iota(jnp.int32, x.shape, 0)
    lg = jnp.sum(jnp.where(lab[None, :, :] == c, x, 0.0), axis=0)
    return lse - lg


def _ce_stats_block(logits_ref, labels_ref, out_ref, acc_ref):
    i = pl.program_id(0) * pl.num_programs(1) + pl.program_id(1)

    @pl.when(i == 0)
    def _init():
        acc_ref[...] = jnp.zeros_like(acc_ref)

    l = _ce_loss(logits_ref, labels_ref)
    keep = l > jnp.float32(_THRESH)
    acc_ref[0] += jnp.where(keep, l, 0.0)
    acc_ref[1] += keep.astype(jnp.float32)

    @pl.when(i == pl.num_programs(0) * pl.num_programs(1) - 1)
    def _fin():
        s_keep = jnp.sum(acc_ref[0])
        c_keep = jnp.sum(acc_ref[1])
        out_ref[0] = s_keep
        out_ref[1] = c_keep
        out_ref[2] = s_keep / c_keep


def _ce_block(logits_ref, labels_ref, loss_ref):
    loss_ref[0] = _ce_loss(logits_ref, labels_ref)


def _sc_select(total):
    per_w = total // _NT
    nchunks = per_w // _CHUNK
    mesh = plsc.VectorSubcoreMesh(core_axis_name="c", subcore_axis_name="s")

    @functools.partial(
        pl.kernel, mesh=mesh,
        out_type=(
            jax.ShapeDtypeStruct((_NT, 16), jnp.float32),
            jax.ShapeDtypeStruct((_NT, 16), jnp.float32),
            jax.ShapeDtypeStruct((16,), jnp.int32),
        ),
        scratch_types=[
            pltpu.VMEM((_CHUNK,), jnp.float32),
            pltpu.VMEM((16,), jnp.float32),
            pltpu.VMEM((16,), jnp.float32),
            pltpu.VMEM((16,), jnp.int32),
            pltpu.VMEM((16,), jnp.float32),
            pltpu.SMEM((1,), jnp.int32),
        ],
    )
    def sc_select(x_hbm, sums_hbm, cnts_hbm, t_hbm, buf, sumv, cntv, tv, redv,
                  cnt_s):
        cid = lax.axis_index("c")
        sid = lax.axis_index("s")

        @pl.when(cid == 0)
        def _():
            base = sid * per_w
            zero16i = jnp.zeros((16,), jnp.int32)

            cnt_s[0] = 0
            plsc.subcore_barrier()

            def count_ge(cand):
                candf = lax.bitcast_convert_type(cand, jnp.float32)
                cf = jnp.zeros((16,), jnp.float32) + candf

                def chunk_body(ci, acc):
                    pltpu.sync_copy(
                        x_hbm.at[pl.ds(base + ci * _CHUNK, _CHUNK)], buf)

                    def inner(i, a):
                        v = buf[pl.ds(i * 16, 16)]
                        return a + jnp.where(v >= cf, 1.0, 0.0)

                    return lax.fori_loop(0, _CHUNK // 16, inner, acc)

                acc = lax.fori_loop(0, nchunks, chunk_body,
                                    jnp.zeros((16,), jnp.float32))
                redv[...] = acc
                vv = redv[...]
                s = vv[0]
                for j in range(1, 16):
                    s = s + vv[j]
                return s.astype(jnp.int32)

            def round_body(r, t):
                cand = t + lax.shift_left(jnp.int32(1), 30 - r)
                c_loc = count_ge(cand)
                for m in range(_NT):
                    plsc.fetch_and_add(cnt_s.at[0], c_loc, subcore_id=m)
                plsc.subcore_barrier()
                g = cnt_s[0]
                cnt_s[0] = 0
                plsc.subcore_barrier()
                return jnp.where(g >= _N_MIN, cand, t)

            t = lax.fori_loop(0, 31, round_body, jnp.int32(0))
            tf = jnp.zeros((16,), jnp.float32) + lax.bitcast_convert_type(
                t, jnp.float32)

            def stats_chunk(ci, carry):
                s, c = carry
                pltpu.sync_copy(
                    x_hbm.at[pl.ds(base + ci * _CHUNK, _CHUNK)], buf)

                def inner(i, sc):
                    s2, c2 = sc
                    v = buf[pl.ds(i * 16, 16)]
                    m = v > tf
                    return (s2 + jnp.where(m, v, 0.0),
                            c2 + jnp.where(m, 1.0, 0.0))

                return lax.fori_loop(0, _CHUNK // 16, inner, (s, c))

            s, c = lax.fori_loop(
                0, nchunks, stats_chunk,
                (jnp.zeros((16,), jnp.float32), jnp.zeros((16,), jnp.float32)))
            sumv[...] = s
            cntv[...] = c
            pltpu.sync_copy(sumv, sums_hbm.at[sid])
            pltpu.sync_copy(cntv, cnts_hbm.at[sid])

            @pl.when(sid == 0)
            def _():
                tv[...] = zero16i + t
                pltpu.sync_copy(tv, t_hbm)

    return sc_select


def _topk_branch(logits, labels):
    N, C, H, W = logits.shape
    BH = 256
    loss = pl.pallas_call(
        _ce_block,
        grid=(N, H // BH),
        in_specs=[
            pl.BlockSpec((1, C, BH, W), lambda n, h: (n, 0, h, 0)),
            pl.BlockSpec((1, BH, W), lambda n, h: (n, h, 0)),
        ],
        out_specs=pl.BlockSpec((1, BH, W), lambda n, h: (n, h, 0)),
        out_shape=jax.ShapeDtypeStruct((N, H, W), jnp.float32),
    )(logits, labels)
    flat = loss.reshape(N * H * W)
    sums, cnts, tvec = _sc_select(flat.shape[0])(flat)
    sum_gt = jnp.sum(sums)
    cnt_gt = jnp.sum(cnts)
    tv = jax.lax.bitcast_convert_type(tvec[0], jnp.float32)
    k = jnp.float32(_N_MIN)
    return (sum_gt + (k - cnt_gt) * tv) / k


@jax.jit
def kernel(logits, labels):
    N, C, H, W = logits.shape
    labels = labels.astype(jnp.int32)
    BH = 512
    stats = pl.pallas_call(
        _ce_stats_block,
        grid=(N, H // BH),
        in_specs=[
            pl.BlockSpec((1, C, BH, W), lambda n, h: (n, 0, h, 0)),
            pl.BlockSpec((1, BH, W), lambda n, h: (n, h, 0)),
        ],
        out_specs=pl.BlockSpec(memory_space=pltpu.SMEM),
        out_shape=jax.ShapeDtypeStruct((3,), jnp.float32),
        scratch_shapes=[pltpu.VMEM((2, BH, W), jnp.float32)],
    )(logits, labels)

    return jax.lax.cond(
        stats[1] >= jnp.float32(_N_MIN),
        lambda: stats[2],
        lambda: _topk_branch(logits, labels),
    )


# speedup vs baseline: 1.2305x; 1.2305x over previous
import functools

import jax
import jax.numpy as jnp
from jax.experimental import pallas as pl
from jax.experimental.pallas import tpu as pltpu

_THRESH = 0.35667494393873245
_N_MIN = 16 * 512 * 512 // 16


def _ce_loss(logits_ref, labels_ref):
    x = logits_ref[0]
    lab = labels_ref[0]
    m = jnp.max(x, axis=0)
    s = jnp.sum(jnp.exp(x - m[None, :, :]), axis=0)
    lse = m + jnp.log(s)
    c = jax.lax.broadcasted_iota(jnp.int32, x.shape, 0)
    lg = jnp.sum(jnp.where(lab[None, :, :] == c, x, 0.0), axis=0)
    return lse - lg


def _ce_stats_block(logits_ref, labels_ref, out_ref, acc_ref):
    i = pl.program_id(0) * pl.num_programs(1) + pl.program_id(1)

    @pl.when(i == 0)
    def _init():
        acc_ref[...] = jnp.zeros_like(acc_ref)

    l = _ce_loss(logits_ref, labels_ref)
    keep = l > jnp.float32(_THRESH)
    acc_ref[0] += jnp.where(keep, l, 0.0)
    acc_ref[1] += keep.astype(jnp.float32)

    @pl.when(i == pl.num_programs(0) * pl.num_programs(1) - 1)
    def _fin():
        s_keep = jnp.sum(acc_ref[0])
        c_keep = jnp.sum(acc_ref[1])
        out_ref[0] = s_keep
        out_ref[1] = c_keep
        out_ref[2] = s_keep / c_keep


def _ce_block(logits_ref, labels_ref, loss_ref):
    loss_ref[0] = _ce_loss(logits_ref, labels_ref)


def _select_block(loss_ref, out_ref):
    v = loss_ref[...]
    p = jax.lax.bitcast_convert_type(v, jnp.int32)
    k = jnp.int32(_N_MIN)

    def body(i, t):
        cand = t + jax.lax.shift_left(jnp.int32(1), 30 - i)
        cnt = jnp.sum((p >= cand).astype(jnp.int32))
        return jnp.where(cnt >= k, cand, t)

    t = jax.lax.fori_loop(0, 31, body, jnp.int32(0))
    tv = jax.lax.bitcast_convert_type(t, jnp.float32)

    gt = p > t
    cnt_gt = jnp.sum(gt.astype(jnp.int32))
    sum_gt = jnp.sum(jnp.where(gt, v, 0.0))
    out_ref[0, 0] = (sum_gt + (k - cnt_gt).astype(jnp.float32) * tv) / jnp.float32(_N_MIN)


def _topk_branch(logits, labels):
    N, C, H, W = logits.shape
    BH = 256
    loss = pl.pallas_call(
        _ce_block,
        grid=(N, H // BH),
        in_specs=[
            pl.BlockSpec((1, C, BH, W), lambda n, h: (n, 0, h, 0)),
            pl.BlockSpec((1, BH, W), lambda n, h: (n, h, 0)),
        ],
        out_specs=pl.BlockSpec((1, BH, W), lambda n, h: (n, h, 0)),
        out_shape=jax.ShapeDtypeStruct((N, H, W), jnp.float32),
    )(logits, labels)
    flat = loss.reshape(N * H * W // 1024, 1024)
    out = pl.pallas_call(
        _select_block,
        in_specs=[pl.BlockSpec(flat.shape, lambda: (0, 0))],
        out_specs=pl.BlockSpec(memory_space=pltpu.SMEM),
        out_shape=jax.ShapeDtypeStruct((1, 1), jnp.float32),
    )(flat)
    return out[0, 0]


@jax.jit
def kernel(logits, labels):
    N, C, H, W = logits.shape
    labels = labels.astype(jnp.int32)
    BH = 512
    stats = pl.pallas_call(
        _ce_stats_block,
        grid=(N, H // BH),
        in_specs=[
            pl.BlockSpec((1, C, BH, W), lambda n, h: (n, 0, h, 0)),
            pl.BlockSpec((1, BH, W), lambda n, h: (n, h, 0)),
        ],
        out_specs=pl.BlockSpec(memory_space=pltpu.SMEM),
        out_shape=jax.ShapeDtypeStruct((3,), jnp.float32),
        scratch_shapes=[pltpu.VMEM((2, BH, W), jnp.float32)],
    )(logits, labels)

    return jax.lax.cond(
        stats[1] >= jnp.float32(_N_MIN),
        lambda: stats[2],
        lambda: _topk_branch(logits, labels),
    )
